# Initial kernel scaffold; baseline (speedup 1.0000x reference)
import jax
import jax.numpy as jnp
from jax.experimental import pallas as pl


def kernel(diffusion_step, W1, b1, W2, b2):
    raise NotImplementedError("write your pallas kernel here")


# speedup vs baseline: 1.4047x; 1.4047x over previous
import functools

import jax
import jax.numpy as jnp
from jax import lax
from jax.experimental import pallas as pl
from jax.experimental.pallas import tpu as pltpu
from jax.experimental.pallas import tpu_sc as plsc

_MAX_STEPS = 1000
_BATCH = 16384
_D = 512

_NC = 2
_NS = 16
_NW = _NC * _NS
_ROWS_PER_W = _BATCH // _NW
_CHUNK = 64
_K = _ROWS_PER_W // _CHUNK


def _build_table():
    steps = jnp.arange(_MAX_STEPS, dtype=jnp.float32)[:, None]
    dims = jnp.arange(64, dtype=jnp.float32)[None, :]
    t = steps * 10.0 ** (dims * 4.0 / 63.0)
    return jnp.concatenate([jnp.sin(t), jnp.cos(t)], axis=1)


def _mlp_body(t_ref, w1_ref, b1_ref, w2_ref, b2_ref, o_ref):
    x = t_ref[...]
    h = jnp.dot(x, w1_ref[...], preferred_element_type=jnp.float32) + b1_ref[...]
    h = h * jax.nn.sigmoid(h)
    o = jnp.dot(h, w2_ref[...], preferred_element_type=jnp.float32) + b2_ref[...]
    o_ref[...] = o * jax.nn.sigmoid(o)


def _tc_mlp(table, W1, b1, W2, b2):
    return pl.pallas_call(
        _mlp_body,
        out_shape=jax.ShapeDtypeStruct((_MAX_STEPS, _D), jnp.float32),
    )(table, W1, b1.reshape(1, _D), W2, b2.reshape(1, _D))


def _gather_body(table_hbm, idx_hbm, out_hbm, idx_v, rows0, rows1, sem0, sem1):
    wid = lax.axis_index("s") * _NC + lax.axis_index("c")
    pltpu.sync_copy(idx_hbm.at[pl.ds(wid * _K, _K)], idx_v)
    bufs = (rows0, rows1)
    sems = (sem0, sem1)
    handles = [None, None]
    handles[0] = pltpu.async_copy(table_hbm.at[idx_v.at[0]], bufs[0], sems[0])
    for j in range(_K):
        cur = j % 2
        if j + 1 < _K:
            nxt = (j + 1) % 2
            handles[nxt] = pltpu.async_copy(
                table_hbm.at[idx_v.at[j + 1]], bufs[nxt], sems[nxt])
        handles[cur].wait()
        pltpu.sync_copy(
            bufs[cur],
            out_hbm.at[pl.ds(wid * _ROWS_PER_W + j * _CHUNK, _CHUNK)])


def _sc_gather(final_table, idx2d):
    mesh = plsc.VectorSubcoreMesh(core_axis_name="c", subcore_axis_name="s")
    k = functools.partial(
        pl.kernel,
        mesh=mesh,
        out_type=jax.ShapeDtypeStruct((_BATCH, _D), jnp.float32),
        scratch_types=[
            pltpu.VMEM((_K, _CHUNK), jnp.int32),
            pltpu.VMEM((_CHUNK, _D), jnp.float32),
            pltpu.VMEM((_CHUNK, _D), jnp.float32),
            pltpu.SemaphoreType.DMA,
            pltpu.SemaphoreType.DMA,
        ],
    )(_gather_body)
    return k(final_table, idx2d)


def kernel(diffusion_step, W1, b1, W2, b2):
    table = _build_table()
    final_table = _tc_mlp(table, W1, b1, W2, b2)
    idx2d = diffusion_step.astype(jnp.int32).reshape(_NW * _K, _CHUNK)
    return _sc_gather(final_table, idx2d)


# speedup vs baseline: 1.4284x; 1.0169x over previous
import functools

import jax
import jax.numpy as jnp
from jax import lax
from jax.experimental import pallas as pl
from jax.experimental.pallas import tpu as pltpu
from jax.experimental.pallas import tpu_sc as plsc

_MAX_STEPS = 1000
_BATCH = 16384
_D = 512

_NC = 2
_NS = 16
_NW = _NC * _NS
_ROWS_PER_W = _BATCH // _NW
_CHUNK = 64
_K = _ROWS_PER_W // _CHUNK


def _build_table():
    steps = jnp.arange(_MAX_STEPS, dtype=jnp.float32)[:, None]
    dims = jnp.arange(64, dtype=jnp.float32)[None, :]
    t = steps * 10.0 ** (dims * 4.0 / 63.0)
    return jnp.concatenate([jnp.sin(t), jnp.cos(t)], axis=1)


def _mlp_body(t_ref, w1_ref, b1_ref, w2_ref, b2_ref, o_ref):
    x = t_ref[...]
    h = jnp.dot(x, w1_ref[...], preferred_element_type=jnp.float32) + b1_ref[...]
    h = h * jax.nn.sigmoid(h)
    o = jnp.dot(h, w2_ref[...], preferred_element_type=jnp.float32) + b2_ref[...]
    o_ref[...] = o * jax.nn.sigmoid(o)


def _tc_mlp(table, W1, b1, W2, b2):
    return pl.pallas_call(
        _mlp_body,
        out_shape=jax.ShapeDtypeStruct((_MAX_STEPS, _D), jnp.float32),
    )(table, W1, b1.reshape(1, _D), W2, b2.reshape(1, _D))


_NBUF = 3


def _gather_body(table_hbm, idx_hbm, out_hbm, idx_v,
                 rows0, rows1, rows2, gsem0, gsem1, gsem2,
                 wsem0, wsem1, wsem2):
    wid = lax.axis_index("s") * _NC + lax.axis_index("c")
    base = wid * _ROWS_PER_W
    bufs = (rows0, rows1, rows2)
    gsems = (gsem0, gsem1, gsem2)
    wsems = (wsem0, wsem1, wsem2)
    pltpu.sync_copy(idx_hbm.at[pl.ds(base, _ROWS_PER_W)], idx_v)
    g = [None] * _NBUF
    w = [None] * _NBUF
    for j in range(_K):
        b = j % _NBUF
        if w[b] is not None:
            w[b].wait()
        g[b] = pltpu.async_copy(
            table_hbm.at[idx_v.at[pl.ds(j * _CHUNK, _CHUNK)]],
            bufs[b], gsems[b])
        if j >= _NBUF - 1:
            jj = j - (_NBUF - 1)
            bb = jj % _NBUF
            g[bb].wait()
            w[bb] = pltpu.async_copy(
                bufs[bb], out_hbm.at[pl.ds(base + jj * _CHUNK, _CHUNK)],
                wsems[bb])
    for jj in range(_K - (_NBUF - 1), _K):
        bb = jj % _NBUF
        g[bb].wait()
        w[bb] = pltpu.async_copy(
            bufs[bb], out_hbm.at[pl.ds(base + jj * _CHUNK, _CHUNK)],
            wsems[bb])
    for bb in range(_NBUF):
        if w[bb] is not None:
            w[bb].wait()


def _sc_gather(final_table, idx):
    mesh = plsc.VectorSubcoreMesh(core_axis_name="c", subcore_axis_name="s")
    k = functools.partial(
        pl.kernel,
        mesh=mesh,
        out_type=jax.ShapeDtypeStruct((_BATCH, _D), jnp.float32),
        scratch_types=[
            pltpu.VMEM((_ROWS_PER_W,), jnp.int32),
            pltpu.VMEM((_CHUNK, _D), jnp.float32),
            pltpu.VMEM((_CHUNK, _D), jnp.float32),
            pltpu.VMEM((_CHUNK, _D), jnp.float32),
            pltpu.SemaphoreType.DMA,
            pltpu.SemaphoreType.DMA,
            pltpu.SemaphoreType.DMA,
            pltpu.SemaphoreType.DMA,
            pltpu.SemaphoreType.DMA,
            pltpu.SemaphoreType.DMA,
        ],
    )(_gather_body)
    return k(final_table, idx)


def kernel(diffusion_step, W1, b1, W2, b2):
    table = _build_table()
    final_table = _tc_mlp(table, W1, b1, W2, b2)
    return _sc_gather(final_table, diffusion_step.astype(jnp.int32))


# speedup vs baseline: 1.4312x; 1.0020x over previous
import functools

import jax
import jax.numpy as jnp
from jax import lax
from jax.experimental import pallas as pl
from jax.experimental.pallas import tpu as pltpu
from jax.experimental.pallas import tpu_sc as plsc

_MAX_STEPS = 1000
_BATCH = 16384
_D = 512

_NC = 2
_NS = 16
_NW = _NC * _NS
_ROWS_PER_W = _BATCH // _NW
_CHUNK = 64
_K = _ROWS_PER_W // _CHUNK


_TPAD = 1024


def _build_table():
    steps = jnp.arange(_TPAD, dtype=jnp.float32)[:, None]
    dims = jnp.arange(64, dtype=jnp.float32)[None, :]
    t = steps * 10.0 ** (dims * 4.0 / 63.0)
    return jnp.concatenate([jnp.sin(t), jnp.cos(t)], axis=1)


_NCOPY = 2


def _mlp_body(t_ref, w1_ref, b1_ref, w2_ref, b2_ref, o_ref):
    x = t_ref[...]
    h = jnp.dot(x, w1_ref[...], preferred_element_type=jnp.float32) + b1_ref[...]
    h = h * jax.nn.sigmoid(h)
    o = jnp.dot(h, w2_ref[...], preferred_element_type=jnp.float32) + b2_ref[...]
    o = o * jax.nn.sigmoid(o)
    for c in range(_NCOPY):
        o_ref[pl.ds(c * _TPAD, _TPAD), :] = o


def _tc_mlp(table, W1, b1, W2, b2):
    return pl.pallas_call(
        _mlp_body,
        out_shape=jax.ShapeDtypeStruct((_NCOPY * _TPAD, _D), jnp.float32),
    )(table, W1, b1.reshape(1, _D), W2, b2.reshape(1, _D))


_NBUF = 2


def _gather_body(table_hbm, idx_hbm, out_hbm, idx_v,
                 rows0, rows1, gsem0, gsem1, wsem0, wsem1):
    sid = lax.axis_index("s")
    cid = lax.axis_index("c")
    wid = sid * _NC + cid
    base = wid * _ROWS_PER_W
    bufs = (rows0, rows1)
    gsems = (gsem0, gsem1)
    wsems = (wsem0, wsem1)
    pltpu.sync_copy(idx_hbm.at[pl.ds(base, _ROWS_PER_W)], idx_v)
    off = (wid % _NCOPY) * _TPAD
    offv = jnp.full((16,), off, jnp.int32)
    for i in range(_ROWS_PER_W // 16):
        idx_v[pl.ds(i * 16, 16)] = idx_v[pl.ds(i * 16, 16)] + offv
    g = [None] * _NBUF
    w = [None] * _NBUF
    for j in range(_K):
        b = j % _NBUF
        if w[b] is not None:
            w[b].wait()
        g[b] = pltpu.async_copy(
            table_hbm.at[idx_v.at[pl.ds(j * _CHUNK, _CHUNK)]],
            bufs[b], gsems[b])
        if j >= _NBUF - 1:
            jj = j - (_NBUF - 1)
            bb = jj % _NBUF
            g[bb].wait()
            w[bb] = pltpu.async_copy(
                bufs[bb], out_hbm.at[pl.ds(base + jj * _CHUNK, _CHUNK)],
                wsems[bb])
    for jj in range(_K - (_NBUF - 1), _K):
        bb = jj % _NBUF
        g[bb].wait()
        w[bb] = pltpu.async_copy(
            bufs[bb], out_hbm.at[pl.ds(base + jj * _CHUNK, _CHUNK)],
            wsems[bb])
    for bb in range(_NBUF):
        if w[bb] is not None:
            w[bb].wait()


def _sc_gather(final_table, idx):
    mesh = plsc.VectorSubcoreMesh(core_axis_name="c", subcore_axis_name="s")
    k = functools.partial(
        pl.kernel,
        mesh=mesh,
        out_type=jax.ShapeDtypeStruct((_BATCH, _D), jnp.float32),
        scratch_types=[
            pltpu.VMEM((_ROWS_PER_W,), jnp.int32),
            pltpu.VMEM((_CHUNK, _D), jnp.float32),
            pltpu.VMEM((_CHUNK, _D), jnp.float32),
            pltpu.SemaphoreType.DMA,
            pltpu.SemaphoreType.DMA,
            pltpu.SemaphoreType.DMA,
            pltpu.SemaphoreType.DMA,
        ],
    )(_gather_body)
    return k(final_table, idx)


def kernel(diffusion_step, W1, b1, W2, b2):
    table = _build_table()
    final_table = _tc_mlp(table, W1, b1, W2, b2)
    return _sc_gather(final_table, diffusion_step.astype(jnp.int32))
